# baseline (speedup 1.0000x reference)
import functools

import jax
import jax.numpy as jnp
from jax.experimental import pallas as pl
from jax.experimental.pallas import tpu as pltpu

_T = 36
_LQ = 3060
_C = 256
_H = 8
_L = 4
_NCP = 4
_NTP = 2
_TW = 2

_BQ = 3072

_DN = (((0,), (1,)), ((), ()))


def _body(q_ref, x_ref, wq_ref, bq_ref, wv_ref, bv_ref, g_ref,
          val_ref, cso_ref, tso_ref, awc_ref, awt_ref):
    x = x_ref[0]
    val_ref[0] = (
        jax.lax.dot_general(wv_ref[:], x, _DN,
                            preferred_element_type=jnp.float32)
        + bv_ref[:]
    )
    q = q_ref[0]
    p = jax.lax.dot_general(wq_ref[:], q, _DN,
                            preferred_element_type=jnp.float32) + bq_ref[:]
    cso_ref[0] = p[:256, :]
    tso_ref[0] = p[256:512, :]
    caw = p[512:640, :]
    taw = p[640:768, :]
    m = jnp.max(jnp.maximum(caw, taw), axis=0, keepdims=True)
    ea = jnp.exp(caw - m)
    eb = jnp.exp(taw - m)
    s = jnp.dot(g_ref[:], ea + eb, preferred_element_type=jnp.float32)
    r = 1.0 / s
    awc_ref[0] = ea * r
    awt_ref[0] = eb * r


@functools.partial(jax.jit)
def kernel(query, input_flatten, W_so, b_so, W_aw, b_aw, W_tso, b_tso,
           W_taw, b_taw, W_vp, b_vp):
    Tn, Lq, Cd = query.shape
    _, Lin, _ = input_flatten.shape

    wq = jnp.concatenate([W_so, W_tso, W_aw, W_taw], axis=1)
    bq = jnp.concatenate([b_so, b_tso, b_aw, b_taw])[:, None]
    bv = b_vp[:, None]
    lane = jnp.arange(128) // 16
    g = (lane[:, None] == lane[None, :]).astype(jnp.float32)

    nbq = pl.cdiv(Lq, _BQ)
    grid = (Tn, nbq)
    act_spec = pl.BlockSpec((1, _BQ, Cd), lambda t, j: (t, j, 0))
    out_spec = lambda f: pl.BlockSpec((1, f, _BQ), lambda t, j: (t, 0, j))
    full_spec = lambda a, b: pl.BlockSpec((a, b), lambda t, j: (0, 0))

    outs = pl.pallas_call(
        _body,
        grid=grid,
        in_specs=[
            act_spec,
            act_spec,
            full_spec(Cd, 768),
            full_spec(768, 1),
            full_spec(Cd, Cd),
            full_spec(Cd, 1),
            full_spec(128, 128),
        ],
        out_specs=[
            out_spec(Cd),
            out_spec(Cd),
            out_spec(Cd),
            out_spec(128),
            out_spec(128),
        ],
        out_shape=[
            jax.ShapeDtypeStruct((Tn, Cd, _BQ), jnp.float32),
            jax.ShapeDtypeStruct((Tn, Cd, _BQ), jnp.float32),
            jax.ShapeDtypeStruct((Tn, Cd, _BQ), jnp.float32),
            jax.ShapeDtypeStruct((Tn, 128, _BQ), jnp.float32),
            jax.ShapeDtypeStruct((Tn, 128, _BQ), jnp.float32),
        ],
        compiler_params=pltpu.CompilerParams(
            dimension_semantics=("parallel", "parallel"),
        ),
    )(query, input_flatten, wq, bq, W_vp, bv, g)

    val_t, cso_t, tso_t, awc_t, awt_t = (o[..., :Lq] for o in outs)
    value = val_t.reshape(Tn, _H, Cd // _H, Lin).transpose(0, 3, 1, 2)
    cso = cso_t.reshape(Tn, _H, _L, _NCP, 2, Lq).transpose(0, 5, 1, 2, 3, 4)
    tso = tso_t.reshape(Tn, _H, _TW * _L, _NTP, 2, Lq).transpose(0, 5, 1, 2, 3, 4)
    aw_curr = awc_t.reshape(Tn, _H, _L, _NCP, Lq).transpose(0, 4, 1, 2, 3)
    aw_temp = awt_t.reshape(Tn, _H, _TW * _L, _NTP, Lq).transpose(0, 4, 1, 2, 3)
    return (value, cso, tso, aw_curr, aw_temp)
